# Initial kernel scaffold; baseline (speedup 1.0000x reference)
import jax
import jax.numpy as jnp
from jax.experimental import pallas as pl


def kernel(positions, colors, markers, anchors, n_jumps, targets, W_pre1, b_pre1, W_pre2, b_pre2, W_msg1, b_msg1, W_msg2, b_msg2, W_post1, b_post1, W_post2, b_post2, W_out1, b_out1, W_out2, b_out2):
    raise NotImplementedError("write your pallas kernel here")


# speedup vs baseline: 26.6236x; 26.6236x over previous
import jax
import jax.numpy as jnp
from jax.experimental import pallas as pl

_N = 8
_BLK = 512


def _dot(a_bf, w_bf):
    return jnp.dot(a_bf, w_bf, preferred_element_type=jnp.float32)


def _rrn_block(
    pos_ref, col_ref, mar_ref, anc_ref, njp_ref, tgt_ref,
    w_pre1_ref, b_pre1_ref, w_pre2_ref, b_pre2_ref,
    w_msg1_ref, b_msg1_ref, w_msg2_ref, b_msg2_ref,
    w_post1_ref, b_post1_ref, w_post2_ref, b_post2_ref,
    w_out1_ref, b_out1_ref, w_out2_ref, b_out2_ref,
    ce_ref, idx_ref,
):
    blk = ce_ref.shape[0]
    f32, bf16 = jnp.float32, jnp.bfloat16

    iota16 = jax.lax.broadcasted_iota(jnp.int32, (blk, 16), 1)
    iota8 = jax.lax.broadcasted_iota(jnp.int32, (blk, _N), 1)
    anc_oh = (iota16 == anc_ref[...]).astype(f32)
    jmp_oh = (iota8 == njp_ref[...]).astype(f32)

    w_pre1 = w_pre1_ref[...]
    b_pre1 = b_pre1_ref[...]
    w_pre2 = w_pre2_ref[...]
    b_pre2 = b_pre2_ref[...]

    pos = []
    xbf = []
    for i in range(_N):
        p_i = pos_ref[i]
        col_oh = (iota8 == col_ref[i]).astype(f32)
        mar_oh = (iota8 == (mar_ref[i] - 8)).astype(f32)
        xin = jnp.concatenate([p_i, col_oh, mar_oh, anc_oh, jmp_oh], axis=1)
        lin = _dot(xin.astype(bf16), w_pre1) + b_pre1
        x_i = _dot(jnp.maximum(lin, 0.0).astype(bf16), w_pre2) + b_pre2
        pos.append(p_i)
        xbf.append(x_i.astype(bf16))

    dist = {}
    for i in range(_N):
        for j in range(i + 1, _N):
            dif = pos[i] - pos[j]
            d = jnp.sqrt(jnp.sum(dif * dif, axis=1, keepdims=True))
            dist[(i, j)] = d.astype(bf16)

    w_msg1 = w_msg1_ref[...]
    b_msg1 = b_msg1_ref[...]
    w_msg2 = w_msg2_ref[...]
    b_msg2 = b_msg2_ref[...]
    w_post1 = w_post1_ref[...]
    b_post1 = b_post1_ref[...]
    w_post2 = w_post2_ref[...]
    b_post2 = b_post2_ref[...]

    feat = None
    for i in range(_N):
        acc = None
        for j in range(_N):
            if j == i:
                continue
            d = dist[(i, j) if i < j else (j, i)]
            op = jnp.concatenate([xbf[i], xbf[j], d], axis=1)
            h = jnp.maximum(_dot(op, w_msg1) + b_msg1, 0.0)
            msgs = _dot(h.astype(bf16), w_msg2) + b_msg2
            acc = msgs if acc is None else acc + msgs
        pin = jnp.concatenate([acc.astype(bf16), xbf[i]], axis=1)
        u = jnp.maximum(_dot(pin, w_post1) + b_post1, 0.0)
        xp = _dot(u.astype(bf16), w_post2) + b_post2
        feat = xp if feat is None else feat + xp

    h_out = jnp.maximum(_dot(feat.astype(bf16), w_out1_ref[...])
                        + b_out1_ref[...], 0.0)
    logits = _dot(h_out.astype(bf16), w_out2_ref[...]) + b_out2_ref[...]

    mx = jnp.max(logits, axis=1, keepdims=True)
    cand = jnp.where(logits == mx, iota16, 16)
    idx_ref[...] = jnp.min(cand, axis=1, keepdims=True)

    shifted = logits - mx
    lse = jnp.log(jnp.sum(jnp.exp(shifted), axis=1, keepdims=True))
    logp = shifted - lse
    tgt_oh = (iota16 == tgt_ref[...]).astype(f32)
    ce_ref[...] = -jnp.sum(logp * tgt_oh, axis=1, keepdims=True)


def kernel(positions, colors, markers, anchors, n_jumps, targets,
           W_pre1, b_pre1, W_pre2, b_pre2,
           W_msg1, b_msg1, W_msg2, b_msg2,
           W_post1, b_post1, W_post2, b_post2,
           W_out1, b_out1, W_out2, b_out2):
    bs = positions.shape[0]
    grid = (bs + _BLK - 1) // _BLK
    bsp = grid * _BLK
    padn = bsp - bs

    pos = jnp.pad(positions, ((0, padn), (0, 0), (0, 0))).transpose(1, 0, 2)
    col = jnp.pad(colors, ((0, padn), (0, 0))).transpose(1, 0)[:, :, None]
    mar = jnp.pad(markers, ((0, padn), (0, 0)),
                  constant_values=8).transpose(1, 0)[:, :, None]
    anc = jnp.pad(anchors, (0, padn))[:, None]
    njp = jnp.pad(n_jumps, (0, padn))[:, None]
    tgt = jnp.pad(targets, (0, padn))[:, None].astype(jnp.int32)

    f32 = jnp.float32
    bf16 = jnp.bfloat16
    row = lambda b: b.reshape(1, -1)
    wb = lambda w: w.astype(bf16)

    def spec(shape, imap):
        return pl.BlockSpec(shape, imap)

    batch3 = lambda shape: spec(shape, lambda b: (0, b, 0))
    batch2 = lambda shape: spec(shape, lambda b: (b, 0))
    full = lambda shape: spec(shape, lambda b: (0, 0))

    operands = (
        pos, col, mar, anc, njp, tgt,
        wb(W_pre1), row(b_pre1), wb(W_pre2), row(b_pre2),
        wb(W_msg1), row(b_msg1), wb(W_msg2), row(b_msg2),
        wb(W_post1), row(b_post1), wb(W_post2), row(b_post2),
        wb(W_out1), row(b_out1), wb(W_out2), row(b_out2),
    )
    in_specs = [
        batch3((_N, _BLK, 2)), batch3((_N, _BLK, 1)), batch3((_N, _BLK, 1)),
        batch2((_BLK, 1)), batch2((_BLK, 1)), batch2((_BLK, 1)),
        full((42, 128)), full((1, 128)), full((128, 128)), full((1, 128)),
        full((257, 128)), full((1, 128)), full((128, 128)), full((1, 128)),
        full((256, 128)), full((1, 128)), full((128, 128)), full((1, 128)),
        full((128, 128)), full((1, 128)), full((128, 16)), full((1, 16)),
    ]
    out_specs = [batch2((_BLK, 1)), batch2((_BLK, 1))]
    out_shape = [
        jax.ShapeDtypeStruct((bsp, 1), f32),
        jax.ShapeDtypeStruct((bsp, 1), jnp.int32),
    ]

    ce_pad, idx_pad = pl.pallas_call(
        _rrn_block,
        grid=(grid,),
        in_specs=in_specs,
        out_specs=out_specs,
        out_shape=out_shape,
    )(*operands)

    ce = ce_pad[:bs, 0]
    out = idx_pad[:bs, 0]
    loss = jnp.mean(ce / jnp.log(2.0))
    return (loss, out)


# speedup vs baseline: 28.5154x; 1.0711x over previous
import jax
import jax.numpy as jnp
from jax.experimental import pallas as pl

_N = 8
_BLK = 512


def _dot(a_bf, w_bf):
    return jnp.dot(a_bf, w_bf, preferred_element_type=jnp.float32)


def _rrn_block(
    pos_ref, pos_t_ref, col_ref, mar_ref, anc_ref, njp_ref, tgt_ref,
    w_pre1_ref, b_pre1_ref, w_pre2_ref, b_pre2_ref,
    w_msg1_ref, b_msg1_ref, w_msg2_ref, b_msg2_ref,
    w_post1_ref, b_post1_ref, w_post2_ref, b_post2_ref,
    w_out1_ref, b_out1_ref, w_out2_ref, b_out2_ref,
    ce_ref, idx_ref,
):
    blk = ce_ref.shape[0]
    f32, bf16 = jnp.float32, jnp.bfloat16

    iota16 = jax.lax.broadcasted_iota(jnp.int32, (blk, 16), 1)
    iota8 = jax.lax.broadcasted_iota(jnp.int32, (blk, _N), 1)
    anc_oh = (iota16 == anc_ref[...]).astype(f32)
    jmp_oh = (iota8 == njp_ref[...]).astype(f32)

    w_pre1 = w_pre1_ref[...]
    b_pre1 = b_pre1_ref[...]
    w_pre2 = w_pre2_ref[...]
    b_pre2 = b_pre2_ref[...]

    xbf = []
    for i in range(_N):
        p_i = pos_ref[i]
        col_oh = (iota8 == col_ref[i]).astype(f32)
        mar_oh = (iota8 == (mar_ref[i] - 8)).astype(f32)
        xin = jnp.concatenate([p_i, col_oh, mar_oh, anc_oh, jmp_oh], axis=1)
        lin = _dot(xin.astype(bf16), w_pre1) + b_pre1
        x_i = _dot(jnp.maximum(lin, 0.0).astype(bf16), w_pre2) + b_pre2
        xbf.append(x_i.astype(bf16))

    px = pos_t_ref[0]
    py = pos_t_ref[1]
    drows = []
    pair_col = {}
    for i in range(_N):
        for j in range(i + 1, _N):
            dx = px[i:i + 1, :] - px[j:j + 1, :]
            dy = py[i:i + 1, :] - py[j:j + 1, :]
            pair_col[(i, j)] = len(drows)
            drows.append(jnp.sqrt(dx * dx + dy * dy))
    dcols = jnp.concatenate(drows, axis=0).T.astype(bf16)
    dist = {p: dcols[:, k:k + 1] for p, k in pair_col.items()}

    w_msg1 = w_msg1_ref[...]
    b_msg1 = b_msg1_ref[...]
    w_msg2 = w_msg2_ref[...]
    b_msg2 = b_msg2_ref[...]
    w_post1 = w_post1_ref[...]
    b_post1 = b_post1_ref[...]
    w_post2 = w_post2_ref[...]
    b_post2 = b_post2_ref[...]

    feat = None
    for i in range(_N):
        acc = None
        for j in range(_N):
            if j == i:
                continue
            d = dist[(i, j) if i < j else (j, i)]
            op = jnp.concatenate([xbf[i], xbf[j], d], axis=1)
            h = jnp.maximum(_dot(op, w_msg1) + b_msg1, 0.0)
            msgs = _dot(h.astype(bf16), w_msg2) + b_msg2
            acc = msgs if acc is None else acc + msgs
        pin = jnp.concatenate([acc.astype(bf16), xbf[i]], axis=1)
        u = jnp.maximum(_dot(pin, w_post1) + b_post1, 0.0)
        xp = _dot(u.astype(bf16), w_post2) + b_post2
        feat = xp if feat is None else feat + xp

    h_out = jnp.maximum(_dot(feat.astype(bf16), w_out1_ref[...])
                        + b_out1_ref[...], 0.0)
    logits = _dot(h_out.astype(bf16), w_out2_ref[...]) + b_out2_ref[...]

    mx = jnp.max(logits, axis=1, keepdims=True)
    cand = jnp.where(logits == mx, iota16, 16)
    idx_ref[...] = jnp.min(cand, axis=1, keepdims=True)

    shifted = logits - mx
    lse = jnp.log(jnp.sum(jnp.exp(shifted), axis=1, keepdims=True))
    logp = shifted - lse
    tgt_oh = (iota16 == tgt_ref[...]).astype(f32)
    ce_ref[...] = -jnp.sum(logp * tgt_oh, axis=1, keepdims=True)


def kernel(positions, colors, markers, anchors, n_jumps, targets,
           W_pre1, b_pre1, W_pre2, b_pre2,
           W_msg1, b_msg1, W_msg2, b_msg2,
           W_post1, b_post1, W_post2, b_post2,
           W_out1, b_out1, W_out2, b_out2):
    bs = positions.shape[0]
    grid = (bs + _BLK - 1) // _BLK
    bsp = grid * _BLK
    padn = bsp - bs

    pos = jnp.pad(positions, ((0, padn), (0, 0), (0, 0))).transpose(1, 0, 2)
    pos_t = jnp.pad(positions, ((0, padn), (0, 0), (0, 0))).transpose(2, 1, 0)
    col = jnp.pad(colors, ((0, padn), (0, 0))).transpose(1, 0)[:, :, None]
    mar = jnp.pad(markers, ((0, padn), (0, 0)),
                  constant_values=8).transpose(1, 0)[:, :, None]
    anc = jnp.pad(anchors, (0, padn))[:, None]
    njp = jnp.pad(n_jumps, (0, padn))[:, None]
    tgt = jnp.pad(targets, (0, padn))[:, None].astype(jnp.int32)

    f32 = jnp.float32
    bf16 = jnp.bfloat16
    row = lambda b: b.reshape(1, -1)
    wb = lambda w: w.astype(bf16)

    def spec(shape, imap):
        return pl.BlockSpec(shape, imap)

    batch3 = lambda shape: spec(shape, lambda b: (0, b, 0))
    batch2 = lambda shape: spec(shape, lambda b: (b, 0))
    full = lambda shape: spec(shape, lambda b: (0, 0))

    operands = (
        pos, pos_t, col, mar, anc, njp, tgt,
        wb(W_pre1), row(b_pre1), wb(W_pre2), row(b_pre2),
        wb(W_msg1), row(b_msg1), wb(W_msg2), row(b_msg2),
        wb(W_post1), row(b_post1), wb(W_post2), row(b_post2),
        wb(W_out1), row(b_out1), wb(W_out2), row(b_out2),
    )
    in_specs = [
        batch3((_N, _BLK, 2)), spec((2, _N, _BLK), lambda b: (0, 0, b)),
        batch3((_N, _BLK, 1)), batch3((_N, _BLK, 1)),
        batch2((_BLK, 1)), batch2((_BLK, 1)), batch2((_BLK, 1)),
        full((42, 128)), full((1, 128)), full((128, 128)), full((1, 128)),
        full((257, 128)), full((1, 128)), full((128, 128)), full((1, 128)),
        full((256, 128)), full((1, 128)), full((128, 128)), full((1, 128)),
        full((128, 128)), full((1, 128)), full((128, 16)), full((1, 16)),
    ]
    out_specs = [batch2((_BLK, 1)), batch2((_BLK, 1))]
    out_shape = [
        jax.ShapeDtypeStruct((bsp, 1), f32),
        jax.ShapeDtypeStruct((bsp, 1), jnp.int32),
    ]

    ce_pad, idx_pad = pl.pallas_call(
        _rrn_block,
        grid=(grid,),
        in_specs=in_specs,
        out_specs=out_specs,
        out_shape=out_shape,
    )(*operands)

    ce = ce_pad[:bs, 0]
    out = idx_pad[:bs, 0]
    loss = jnp.mean(ce / jnp.log(2.0))
    return (loss, out)


# speedup vs baseline: 30.7408x; 1.0780x over previous
import jax
import jax.numpy as jnp
from jax.experimental import pallas as pl
from jax.experimental.pallas import tpu as pltpu

_N = 8
_BLK = 512


def _dot(a_bf, w_bf):
    return jnp.dot(a_bf, w_bf, preferred_element_type=jnp.float32)


def _rrn_block(
    pos_ref, pos_t_ref, col_ref, mar_ref, anc_ref, njp_ref, tgt_ref,
    w_pre1_ref, b_pre1_ref, w_pre2_ref, b_pre2_ref,
    w_msg1_ref, b_msg1_ref, w_msg2_ref, b_msg2_ref,
    w_post1_ref, b_post1_ref, w_post2_ref, b_post2_ref,
    w_out1_ref, b_out1_ref, w_out2_ref, b_out2_ref,
    ce_ref, idx_ref,
    msc0, msc1, psc,
):
    blk = ce_ref.shape[0]
    f32, bf16 = jnp.float32, jnp.bfloat16

    iota16 = jax.lax.broadcasted_iota(jnp.int32, (blk, 16), 1)
    iota8 = jax.lax.broadcasted_iota(jnp.int32, (blk, _N), 1)
    anc_oh = (iota16 == anc_ref[...]).astype(f32)
    jmp_oh = (iota8 == njp_ref[...]).astype(f32)

    w_pre1 = w_pre1_ref[...]
    b_pre1 = b_pre1_ref[...]
    w_pre2 = w_pre2_ref[...]
    b_pre2 = b_pre2_ref[...]

    xins = []
    for i in range(_N):
        p_i = pos_ref[i]
        col_oh = (iota8 == col_ref[i]).astype(f32)
        mar_oh = (iota8 == (mar_ref[i] - 8)).astype(f32)
        xin = jnp.concatenate([p_i, col_oh, mar_oh, anc_oh, jmp_oh], axis=1)
        xins.append(xin.astype(bf16))
    xin_all = jnp.concatenate(xins, axis=0)
    lin = _dot(xin_all, w_pre1) + b_pre1
    x_all = _dot(jnp.maximum(lin, 0.0).astype(bf16), w_pre2) + b_pre2
    xbf_all = x_all.astype(bf16)
    xbf = [xbf_all[i * blk:(i + 1) * blk] for i in range(_N)]

    px = pos_t_ref[0]
    py = pos_t_ref[1]
    drows = []
    pair_col = {}
    for i in range(_N):
        for j in range(i + 1, _N):
            dx = px[i:i + 1, :] - px[j:j + 1, :]
            dy = py[i:i + 1, :] - py[j:j + 1, :]
            pair_col[(i, j)] = len(drows)
            drows.append(jnp.sqrt(dx * dx + dy * dy))
    dcols = jnp.concatenate(drows, axis=0).T.astype(bf16)
    dist = {p: dcols[:, k:k + 1] for p, k in pair_col.items()}

    w_msg1 = w_msg1_ref[...]
    b_msg1 = b_msg1_ref[...]
    w_msg2 = w_msg2_ref[...]
    b_msg2 = b_msg2_ref[...]
    w_post1 = w_post1_ref[...]
    b_post1 = b_post1_ref[...]
    w_post2 = w_post2_ref[...]
    b_post2 = b_post2_ref[...]

    for i in range(_N):
        msc = msc0 if i % 2 == 0 else msc1
        js = [j for j in range(_N) if j != i]
        for k, j in enumerate(js):
            r0 = k * blk
            msc[r0:r0 + blk, 0:128] = xbf[i]
            msc[r0:r0 + blk, 128:256] = xbf[j]
            msc[r0:r0 + blk, 256:257] = dist[(i, j) if i < j else (j, i)]
        h = jnp.maximum(_dot(msc[...], w_msg1) + b_msg1, 0.0)
        msgs = _dot(h.astype(bf16), w_msg2) + b_msg2
        acc = msgs[0:blk]
        for k in range(1, _N - 1):
            acc = acc + msgs[k * blk:(k + 1) * blk]
        psc[i * blk:(i + 1) * blk, 0:128] = acc.astype(bf16)
        psc[i * blk:(i + 1) * blk, 128:256] = xbf[i]
    u = jnp.maximum(_dot(psc[...], w_post1) + b_post1, 0.0)
    xp = _dot(u.astype(bf16), w_post2) + b_post2
    feat = xp[0:blk]
    for i in range(1, _N):
        feat = feat + xp[i * blk:(i + 1) * blk]

    h_out = jnp.maximum(_dot(feat.astype(bf16), w_out1_ref[...])
                        + b_out1_ref[...], 0.0)
    logits = _dot(h_out.astype(bf16), w_out2_ref[...]) + b_out2_ref[...]

    mx = jnp.max(logits, axis=1, keepdims=True)
    cand = jnp.where(logits == mx, iota16, 16)
    idx_ref[...] = jnp.min(cand, axis=1, keepdims=True)

    shifted = logits - mx
    lse = jnp.log(jnp.sum(jnp.exp(shifted), axis=1, keepdims=True))
    logp = shifted - lse
    tgt_oh = (iota16 == tgt_ref[...]).astype(f32)
    ce_ref[...] = -jnp.sum(logp * tgt_oh, axis=1, keepdims=True)


def kernel(positions, colors, markers, anchors, n_jumps, targets,
           W_pre1, b_pre1, W_pre2, b_pre2,
           W_msg1, b_msg1, W_msg2, b_msg2,
           W_post1, b_post1, W_post2, b_post2,
           W_out1, b_out1, W_out2, b_out2):
    bs = positions.shape[0]
    grid = (bs + _BLK - 1) // _BLK
    bsp = grid * _BLK
    padn = bsp - bs

    pos = jnp.pad(positions, ((0, padn), (0, 0), (0, 0))).transpose(1, 0, 2)
    pos_t = jnp.pad(positions, ((0, padn), (0, 0), (0, 0))).transpose(2, 1, 0)
    col = jnp.pad(colors, ((0, padn), (0, 0))).transpose(1, 0)[:, :, None]
    mar = jnp.pad(markers, ((0, padn), (0, 0)),
                  constant_values=8).transpose(1, 0)[:, :, None]
    anc = jnp.pad(anchors, (0, padn))[:, None]
    njp = jnp.pad(n_jumps, (0, padn))[:, None]
    tgt = jnp.pad(targets, (0, padn))[:, None].astype(jnp.int32)

    f32 = jnp.float32
    bf16 = jnp.bfloat16
    row = lambda b: b.reshape(1, -1)
    wb = lambda w: w.astype(bf16)

    def spec(shape, imap):
        return pl.BlockSpec(shape, imap)

    batch3 = lambda shape: spec(shape, lambda b: (0, b, 0))
    batch2 = lambda shape: spec(shape, lambda b: (b, 0))
    full = lambda shape: spec(shape, lambda b: (0, 0))

    operands = (
        pos, pos_t, col, mar, anc, njp, tgt,
        wb(W_pre1), row(b_pre1), wb(W_pre2), row(b_pre2),
        wb(W_msg1), row(b_msg1), wb(W_msg2), row(b_msg2),
        wb(W_post1), row(b_post1), wb(W_post2), row(b_post2),
        wb(W_out1), row(b_out1), wb(W_out2), row(b_out2),
    )
    in_specs = [
        batch3((_N, _BLK, 2)), spec((2, _N, _BLK), lambda b: (0, 0, b)),
        batch3((_N, _BLK, 1)), batch3((_N, _BLK, 1)),
        batch2((_BLK, 1)), batch2((_BLK, 1)), batch2((_BLK, 1)),
        full((42, 128)), full((1, 128)), full((128, 128)), full((1, 128)),
        full((257, 128)), full((1, 128)), full((128, 128)), full((1, 128)),
        full((256, 128)), full((1, 128)), full((128, 128)), full((1, 128)),
        full((128, 128)), full((1, 128)), full((128, 16)), full((1, 16)),
    ]
    out_specs = [batch2((_BLK, 1)), batch2((_BLK, 1))]
    out_shape = [
        jax.ShapeDtypeStruct((bsp, 1), f32),
        jax.ShapeDtypeStruct((bsp, 1), jnp.int32),
    ]

    ce_pad, idx_pad = pl.pallas_call(
        _rrn_block,
        grid=(grid,),
        in_specs=in_specs,
        out_specs=out_specs,
        out_shape=out_shape,
        scratch_shapes=[
            pltpu.VMEM(((_N - 1) * _BLK, 257), bf16),
            pltpu.VMEM(((_N - 1) * _BLK, 257), bf16),
            pltpu.VMEM((_N * _BLK, 256), bf16),
        ],
    )(*operands)

    ce = ce_pad[:bs, 0]
    out = idx_pad[:bs, 0]
    loss = jnp.mean(ce / jnp.log(2.0))
    return (loss, out)


# speedup vs baseline: 34.1948x; 1.1124x over previous
import jax
import jax.numpy as jnp
from jax.experimental import pallas as pl
from jax.experimental.pallas import tpu as pltpu

_N = 8
_BLK = 512


def _dot(a_bf, w_bf):
    return jnp.dot(a_bf, w_bf, preferred_element_type=jnp.float32)


def _rrn_block(
    posr_ref, pos_t_ref, col_ref, mar_ref, anc_ref, njp_ref, tgt_ref,
    w_pre1_ref, b_pre1_ref, w_pre2_ref, b_pre2_ref,
    w_msg1_ref, b_msg1_ref, w_msg2_ref, b_msg2_ref,
    w_post1_ref, b_post1_ref, w_post2_ref, b_post2_ref,
    w_out1_ref, b_out1_ref, w_out2_ref, b_out2_ref,
    ce_ref, idx_ref,
    msc0, msc1, psc,
):
    blk = ce_ref.shape[0]
    f32, bf16 = jnp.float32, jnp.bfloat16

    iota16 = jax.lax.broadcasted_iota(jnp.int32, (blk, 16), 1)
    iota8 = jax.lax.broadcasted_iota(jnp.int32, (blk, _N), 1)
    anc_oh = (iota16 == anc_ref[...]).astype(f32)
    jmp_oh = (iota8 == njp_ref[...]).astype(f32)

    w_pre1 = w_pre1_ref[...]
    b_pre1 = b_pre1_ref[...]
    w_pre2 = w_pre2_ref[...]
    b_pre2 = b_pre2_ref[...]

    xins = []
    for i in range(_N):
        p_i = posr_ref[:, 2 * i:2 * i + 2]
        col_oh = (iota8 == col_ref[:, i:i + 1]).astype(f32)
        mar_oh = (iota8 == (mar_ref[:, i:i + 1] - 8)).astype(f32)
        xin = jnp.concatenate([p_i, col_oh, mar_oh, anc_oh, jmp_oh], axis=1)
        xins.append(xin.astype(bf16))
    xin_all = jnp.concatenate(xins, axis=0)
    lin = _dot(xin_all, w_pre1) + b_pre1
    x_all = _dot(jnp.maximum(lin, 0.0).astype(bf16), w_pre2) + b_pre2
    xbf_all = x_all.astype(bf16)
    xbf = [xbf_all[i * blk:(i + 1) * blk] for i in range(_N)]

    px = pos_t_ref[0]
    py = pos_t_ref[1]
    drows = []
    pair_col = {}
    for i in range(_N):
        for j in range(i + 1, _N):
            dx = px[i:i + 1, :] - px[j:j + 1, :]
            dy = py[i:i + 1, :] - py[j:j + 1, :]
            pair_col[(i, j)] = len(drows)
            drows.append(jnp.sqrt(dx * dx + dy * dy))
    dcols = jnp.concatenate(drows, axis=0).T.astype(bf16)
    dist = {p: dcols[:, k:k + 1] for p, k in pair_col.items()}

    w_msg1 = w_msg1_ref[...]
    b_msg1 = b_msg1_ref[...]
    w_msg2 = w_msg2_ref[...]
    b_msg2 = b_msg2_ref[...]
    w_post1 = w_post1_ref[...]
    b_post1 = b_post1_ref[...]
    w_post2 = w_post2_ref[...]
    b_post2 = b_post2_ref[...]

    for i in range(_N):
        msc = msc0 if i % 2 == 0 else msc1
        js = [j for j in range(_N) if j != i]
        for k, j in enumerate(js):
            r0 = k * blk
            msc[r0:r0 + blk, 0:128] = xbf[i]
            msc[r0:r0 + blk, 128:256] = xbf[j]
            msc[r0:r0 + blk, 256:257] = dist[(i, j) if i < j else (j, i)]
        h = jnp.maximum(_dot(msc[...], w_msg1) + b_msg1, 0.0)
        msgs = _dot(h.astype(bf16), w_msg2) + b_msg2
        acc = msgs[0:blk]
        for k in range(1, _N - 1):
            acc = acc + msgs[k * blk:(k + 1) * blk]
        psc[i * blk:(i + 1) * blk, 0:128] = acc.astype(bf16)
        psc[i * blk:(i + 1) * blk, 128:256] = xbf[i]
    u = jnp.maximum(_dot(psc[...], w_post1) + b_post1, 0.0)
    xp = _dot(u.astype(bf16), w_post2) + b_post2
    feat = xp[0:blk]
    for i in range(1, _N):
        feat = feat + xp[i * blk:(i + 1) * blk]

    h_out = jnp.maximum(_dot(feat.astype(bf16), w_out1_ref[...])
                        + b_out1_ref[...], 0.0)
    logits = _dot(h_out.astype(bf16), w_out2_ref[...]) + b_out2_ref[...]

    mx = jnp.max(logits, axis=1, keepdims=True)
    cand = jnp.where(logits == mx, iota16, 16)
    idx_ref[...] = jnp.min(cand, axis=1, keepdims=True)

    shifted = logits - mx
    lse = jnp.log(jnp.sum(jnp.exp(shifted), axis=1, keepdims=True))
    logp = shifted - lse
    tgt_oh = (iota16 == tgt_ref[...]).astype(f32)
    ce_ref[...] = -jnp.sum(logp * tgt_oh, axis=1, keepdims=True)


def kernel(positions, colors, markers, anchors, n_jumps, targets,
           W_pre1, b_pre1, W_pre2, b_pre2,
           W_msg1, b_msg1, W_msg2, b_msg2,
           W_post1, b_post1, W_post2, b_post2,
           W_out1, b_out1, W_out2, b_out2):
    bs = positions.shape[0]
    grid = (bs + _BLK - 1) // _BLK
    bsp = grid * _BLK
    padn = bsp - bs

    posr = jnp.pad(positions, ((0, padn), (0, 0), (0, 0))).reshape(bsp, 16)
    pos_t = jnp.pad(positions, ((0, padn), (0, 0), (0, 0))).transpose(2, 1, 0)
    col = jnp.pad(colors, ((0, padn), (0, 0)))
    mar = jnp.pad(markers, ((0, padn), (0, 0)), constant_values=8)
    anc = jnp.pad(anchors, (0, padn))[:, None]
    njp = jnp.pad(n_jumps, (0, padn))[:, None]
    tgt = jnp.pad(targets, (0, padn))[:, None].astype(jnp.int32)

    f32 = jnp.float32
    bf16 = jnp.bfloat16
    row = lambda b: b.reshape(1, -1)
    wb = lambda w: w.astype(bf16)

    def spec(shape, imap):
        return pl.BlockSpec(shape, imap)

    batch3 = lambda shape: spec(shape, lambda b: (0, b, 0))
    batch2 = lambda shape: spec(shape, lambda b: (b, 0))
    full = lambda shape: spec(shape, lambda b: (0, 0))

    operands = (
        posr, pos_t, col, mar, anc, njp, tgt,
        wb(W_pre1), row(b_pre1), wb(W_pre2), row(b_pre2),
        wb(W_msg1), row(b_msg1), wb(W_msg2), row(b_msg2),
        wb(W_post1), row(b_post1), wb(W_post2), row(b_post2),
        wb(W_out1), row(b_out1), wb(W_out2), row(b_out2),
    )
    in_specs = [
        batch2((_BLK, 16)), spec((2, _N, _BLK), lambda b: (0, 0, b)),
        batch2((_BLK, _N)), batch2((_BLK, _N)),
        batch2((_BLK, 1)), batch2((_BLK, 1)), batch2((_BLK, 1)),
        full((42, 128)), full((1, 128)), full((128, 128)), full((1, 128)),
        full((257, 128)), full((1, 128)), full((128, 128)), full((1, 128)),
        full((256, 128)), full((1, 128)), full((128, 128)), full((1, 128)),
        full((128, 128)), full((1, 128)), full((128, 16)), full((1, 16)),
    ]
    out_specs = [batch2((_BLK, 1)), batch2((_BLK, 1))]
    out_shape = [
        jax.ShapeDtypeStruct((bsp, 1), f32),
        jax.ShapeDtypeStruct((bsp, 1), jnp.int32),
    ]

    ce_pad, idx_pad = pl.pallas_call(
        _rrn_block,
        grid=(grid,),
        in_specs=in_specs,
        out_specs=out_specs,
        out_shape=out_shape,
        scratch_shapes=[
            pltpu.VMEM(((_N - 1) * _BLK, 257), bf16),
            pltpu.VMEM(((_N - 1) * _BLK, 257), bf16),
            pltpu.VMEM((_N * _BLK, 256), bf16),
        ],
    )(*operands)

    ce = ce_pad[:bs, 0]
    out = idx_pad[:bs, 0]
    loss = jnp.mean(ce / jnp.log(2.0))
    return (loss, out)
